# Initial kernel scaffold; baseline (speedup 1.0000x reference)
import jax
import jax.numpy as jnp
from jax.experimental import pallas as pl


def kernel(indices, weight):
    raise NotImplementedError("write your pallas kernel here")


# speedup vs baseline: 1.8388x; 1.8388x over previous
import functools

import jax
import jax.numpy as jnp
from jax import lax
from jax.experimental import pallas as pl
from jax.experimental.pallas import tpu as pltpu
from jax.experimental.pallas import tpu_sc as plsc

NUM_EMB = 20000
D = 512
BATCH = 4096
HIST = 50
B = BATCH * HIST

_info = plsc.get_sparse_core_info()
_NC, _NS = _info.num_cores, _info.num_subcores
NW = _NC * _NS
B_PER_W = B // NW
CHUNK = 128
NCHUNK = B_PER_W // CHUNK


def _emb_body(table_hbm, idx_hbm, out_hbm, idx_v, rows_v, sem):
    wid = lax.axis_index("s") * _NC + lax.axis_index("c")
    base = wid * B_PER_W
    pltpu.sync_copy(idx_hbm.at[wid], idx_v)

    def body(i, carry):
        pltpu.async_copy(table_hbm.at[idx_v.at[i]], rows_v, sem).wait()
        pltpu.sync_copy(rows_v, out_hbm.at[pl.ds(base + i * CHUNK, CHUNK)])
        return carry

    lax.fori_loop(0, NCHUNK, body, 0)


@jax.jit
def _emb(table, idx3):
    run = pl.kernel(
        _emb_body,
        out_type=jax.ShapeDtypeStruct((B, D), jnp.float32),
        mesh=plsc.VectorSubcoreMesh(core_axis_name="c", subcore_axis_name="s"),
        scratch_types=[
            pltpu.VMEM((NCHUNK, CHUNK), jnp.int32),
            pltpu.VMEM((CHUNK, D), jnp.float32),
            pltpu.SemaphoreType.DMA,
        ],
    )
    return run(table, idx3)


def kernel(indices, weight):
    idx = indices.reshape(-1).astype(jnp.int32).reshape(NW, NCHUNK, CHUNK)
    out = _emb(weight, idx)
    return out.reshape(BATCH, HIST, D)


# speedup vs baseline: 1.8797x; 1.0223x over previous
import functools

import jax
import jax.numpy as jnp
from jax import lax
from jax.experimental import pallas as pl
from jax.experimental.pallas import tpu as pltpu
from jax.experimental.pallas import tpu_sc as plsc

NUM_EMB = 20000
D = 512
BATCH = 4096
HIST = 50
B = BATCH * HIST

_info = plsc.get_sparse_core_info()
_NC, _NS = _info.num_cores, _info.num_subcores
NW = _NC * _NS
B_PER_W = B // NW
CHUNK = 80
NCHUNK = B_PER_W // CHUNK
NPAIR = NCHUNK // 2


def _emb_body(table_hbm, idx_hbm, out_hbm, idx_v, rows0, rows1, sem0, sem1):
    wid = lax.axis_index("s") * _NC + lax.axis_index("c")
    base = wid * B_PER_W
    rows = (rows0, rows1)
    sems = (sem0, sem1)
    pltpu.sync_copy(idx_hbm.at[wid], idx_v)

    pltpu.async_copy(table_hbm.at[idx_v.at[0]], rows0, sem0)
    pltpu.async_copy(table_hbm.at[idx_v.at[1]], rows1, sem1)

    def outer(i, carry):
        for b in range(2):
            c = 2 * i + b
            pltpu.make_async_copy(table_hbm.at[idx_v.at[c]], rows[b], sems[b]).wait()
            pltpu.sync_copy(rows[b], out_hbm.at[pl.ds(base + c * CHUNK, CHUNK)])
            pltpu.async_copy(table_hbm.at[idx_v.at[c + 2]], rows[b], sems[b])
        return carry

    lax.fori_loop(0, NPAIR - 1, outer, 0)

    for b in range(2):
        c = NCHUNK - 2 + b
        pltpu.make_async_copy(table_hbm.at[idx_v.at[c]], rows[b], sems[b]).wait()
        pltpu.sync_copy(rows[b], out_hbm.at[pl.ds(base + c * CHUNK, CHUNK)])


@jax.jit
def _emb(table, idx3):
    run = pl.kernel(
        _emb_body,
        out_type=jax.ShapeDtypeStruct((B, D), jnp.float32),
        mesh=plsc.VectorSubcoreMesh(core_axis_name="c", subcore_axis_name="s"),
        scratch_types=[
            pltpu.VMEM((NCHUNK, CHUNK), jnp.int32),
            pltpu.VMEM((CHUNK, D), jnp.float32),
            pltpu.VMEM((CHUNK, D), jnp.float32),
            pltpu.SemaphoreType.DMA,
            pltpu.SemaphoreType.DMA,
        ],
    )
    return run(table, idx3)


def kernel(indices, weight):
    idx = indices.reshape(-1).astype(jnp.int32).reshape(NW, NCHUNK, CHUNK)
    out = _emb(weight, idx)
    return out.reshape(BATCH, HIST, D)


# speedup vs baseline: 2.8380x; 1.5098x over previous
import functools

import jax
import jax.numpy as jnp
from jax import lax
from jax.experimental import pallas as pl
from jax.experimental.pallas import tpu as pltpu
from jax.experimental.pallas import tpu_sc as plsc

NUM_EMB = 20000
D = 512
BATCH = 4096
HIST = 50
BODY = 48
TAIL = HIST - BODY

_info = plsc.get_sparse_core_info()
_NC, _NS = _info.num_cores, _info.num_subcores
NW = _NC * _NS
B_PER_W = BATCH // NW
NBUF = 4
NGROUP = B_PER_W // NBUF


def _emb_body(table_hbm, idxb_hbm, idxt_hbm, out_hbm, idxb_v, idxt_v,
              a0, a1, a2, a3, t0, t1, t2, t3,
              sa0, sa1, sa2, sa3, st0, st1, st2, st3):
    wid = lax.axis_index("s") * _NC + lax.axis_index("c")
    base = wid * B_PER_W
    bodies = (a0, a1, a2, a3)
    tails = (t0, t1, t2, t3)
    sas = (sa0, sa1, sa2, sa3)
    sts = (st0, st1, st2, st3)
    pltpu.sync_copy(idxb_hbm.at[pl.ds(wid * B_PER_W * BODY, B_PER_W * BODY)], idxb_v)
    pltpu.sync_copy(idxt_hbm.at[wid], idxt_v)

    def gather(b, c):
        pltpu.async_copy(table_hbm.at[idxb_v.at[pl.ds(c * BODY, BODY)]], bodies[b], sas[b])
        pltpu.async_copy(table_hbm.at[idxt_v.at[c]], tails[b], sts[b])

    def wait(b, c):
        pltpu.make_async_copy(table_hbm.at[idxb_v.at[pl.ds(c * BODY, BODY)]], bodies[b], sas[b]).wait()
        pltpu.make_async_copy(table_hbm.at[idxt_v.at[c]], tails[b], sts[b]).wait()

    def writeback(b, c):
        dst = out_hbm.at[base + c]
        pltpu.sync_copy(bodies[b], dst.at[pl.ds(0, BODY)])
        pltpu.sync_copy(tails[b], dst.at[pl.ds(BODY, TAIL)])

    for b in range(NBUF):
        gather(b, b)

    def outer(i, carry):
        for b in range(NBUF):
            c = NBUF * i + b
            wait(b, c)
            writeback(b, c)
            gather(b, c + NBUF)
        return carry

    lax.fori_loop(0, NGROUP - 1, outer, 0)

    for b in range(NBUF):
        c = B_PER_W - NBUF + b
        wait(b, c)
        writeback(b, c)


@jax.jit
def _emb(table, idxb, idxt):
    run = pl.kernel(
        _emb_body,
        out_type=jax.ShapeDtypeStruct((BATCH, HIST, D), jnp.float32),
        mesh=plsc.VectorSubcoreMesh(core_axis_name="c", subcore_axis_name="s"),
        scratch_types=(
            [
                pltpu.VMEM((B_PER_W * BODY,), jnp.int32),
                pltpu.VMEM((B_PER_W, TAIL), jnp.int32),
            ]
            + [pltpu.VMEM((BODY, D), jnp.float32)] * NBUF
            + [pltpu.VMEM((TAIL, D), jnp.float32)] * NBUF
            + [pltpu.SemaphoreType.DMA] * (2 * NBUF)
        ),
    )
    return run(table, idxb, idxt)


def kernel(indices, weight):
    idx = indices.astype(jnp.int32)
    idxb = idx[:, :BODY].reshape(-1)
    idxt = idx[:, BODY:].reshape(NW, B_PER_W, TAIL)
    return _emb(weight, idxb, idxt)


# speedup vs baseline: 5.8878x; 2.0746x over previous
import functools

import jax
import jax.numpy as jnp
from jax import lax
from jax.experimental import pallas as pl
from jax.experimental.pallas import tpu as pltpu
from jax.experimental.pallas import tpu_sc as plsc

NUM_EMB = 20000
D = 512
BATCH = 4096
HIST = 50
B = BATCH * HIST

_info = plsc.get_sparse_core_info()
_NC, _NS = _info.num_cores, _info.num_subcores
NW = _NC * _NS
B_PER_W = B // NW
CHUNK = 80
NCHUNK = B_PER_W // CHUNK
NPAIR = NCHUNK // 2


def _emb_body(table_hbm, idx_hbm, out_hbm, idx_v, rows0, rows1, sem0, sem1):
    wid = lax.axis_index("s") * _NC + lax.axis_index("c")
    base = wid * B_PER_W
    rows = (rows0, rows1)
    sems = (sem0, sem1)
    pltpu.sync_copy(idx_hbm.at[wid], idx_v)

    pltpu.async_copy(table_hbm.at[idx_v.at[0]], rows0, sem0)
    pltpu.async_copy(table_hbm.at[idx_v.at[1]], rows1, sem1)

    def outer(i, carry):
        for b in range(2):
            c = 2 * i + b
            pltpu.make_async_copy(table_hbm.at[idx_v.at[c]], rows[b], sems[b]).wait()
            pltpu.sync_copy(rows[b], out_hbm.at[pl.ds(base + c * CHUNK, CHUNK)])
            pltpu.async_copy(table_hbm.at[idx_v.at[c + 2]], rows[b], sems[b])
        return carry

    lax.fori_loop(0, NPAIR - 1, outer, 0)

    for b in range(2):
        c = NCHUNK - 2 + b
        pltpu.make_async_copy(table_hbm.at[idx_v.at[c]], rows[b], sems[b]).wait()
        pltpu.sync_copy(rows[b], out_hbm.at[pl.ds(base + c * CHUNK, CHUNK)])


@jax.jit
def _emb(table, idx3):
    run = pl.kernel(
        _emb_body,
        out_type=jax.ShapeDtypeStruct((B, D), jnp.float32),
        mesh=plsc.VectorSubcoreMesh(core_axis_name="c", subcore_axis_name="s"),
        scratch_types=[
            pltpu.VMEM((NCHUNK, CHUNK), jnp.int32),
            pltpu.VMEM((CHUNK, D), jnp.float32),
            pltpu.VMEM((CHUNK, D), jnp.float32),
            pltpu.SemaphoreType.DMA,
            pltpu.SemaphoreType.DMA,
        ],
    )
    return run(table, idx3)


def kernel(indices, weight):
    idx3 = indices.astype(jnp.int32).T.reshape(NW, NCHUNK, CHUNK)
    flat = _emb(weight, idx3)
    return flat.reshape(HIST, BATCH, D).transpose(1, 0, 2)
